# Initial kernel scaffold; baseline (speedup 1.0000x reference)
import functools

import jax
import jax.numpy as jnp
from jax import lax
from jax.experimental import pallas as pl
from jax.experimental.pallas import tpu as pltpu
from jax.experimental.pallas import tpu_sc as plsc

_HEADS = 2
_DH = 512
_DF = _HEADS * _DH
_N = 10000
_E = 160000
_DIN = 256
_DEDGE = 7
_NGRAPH = 16

_NW = 32
_NPT = 320
_RP_PAD = 10256
_CE = 32


def _mm_node(h_in, W_l, W_r, fuse_head_mean):
    din = h_in.shape[1] // 2 if fuse_head_mean else h_in.shape[1]
    bn = 1000

    def body(h_ref, wl_ref, wr_ref, ol_ref, or_ref):
        hb = h_ref[...]
        if fuse_head_mean:
            hb = 0.5 * (hb[:, :din] + hb[:, din:])
        ol_ref[...] = jnp.dot(hb, wl_ref[...], preferred_element_type=jnp.float32)
        or_ref[...] = jnp.dot(hb, wr_ref[...], preferred_element_type=jnp.float32)

    return pl.pallas_call(
        body,
        grid=(_N // bn,),
        in_specs=[
            pl.BlockSpec((bn, h_in.shape[1]), lambda i: (i, 0)),
            pl.BlockSpec((din, _DF), lambda i: (0, 0)),
            pl.BlockSpec((din, _DF), lambda i: (0, 0)),
        ],
        out_specs=[
            pl.BlockSpec((bn, _DF), lambda i: (i, 0)),
            pl.BlockSpec((bn, _DF), lambda i: (i, 0)),
        ],
        out_shape=[
            jax.ShapeDtypeStruct((_N, _DF), jnp.float32),
            jax.ShapeDtypeStruct((_N, _DF), jnp.float32),
        ],
    )(h_in, W_l, W_r)


def _mm_edge(edge_attr, W_e):
    be = 8000

    def body(ea_ref, we_ref, o_ref):
        o_ref[...] = jnp.dot(ea_ref[...], we_ref[...],
                             preferred_element_type=jnp.float32)

    return pl.pallas_call(
        body,
        grid=(_E // be,),
        in_specs=[
            pl.BlockSpec((be, _DEDGE), lambda i: (i, 0)),
            pl.BlockSpec((_DEDGE, _DF), lambda i: (0, 0)),
        ],
        out_specs=pl.BlockSpec((be, _DF), lambda i: (i, 0)),
        out_shape=jax.ShapeDtypeStruct((_E, _DF), jnp.float32),
    )(edge_attr, W_e)


def _pool(out2, batch3):
    bn = 1000
    steps = _N // bn

    def body(o_ref, b_ref, res_ref, sums, cnts):
        i = pl.program_id(0)

        @pl.when(i == 0)
        def _():
            sums[...] = jnp.zeros_like(sums)
            cnts[...] = jnp.zeros_like(cnts)

        h = jnp.maximum(0.5 * (o_ref[:, :_DH] + o_ref[:, _DH:]), 0.0)
        b = b_ref[0, 0, :]
        gids = lax.broadcasted_iota(jnp.int32, (_NGRAPH, bn), 0)
        onehot = (gids == b[None, :]).astype(jnp.float32)
        sums[...] += jnp.dot(onehot, h, preferred_element_type=jnp.float32)
        cnts[...] += jnp.broadcast_to(
            jnp.sum(onehot, axis=1, keepdims=True), (_NGRAPH, 128))

        @pl.when(i == steps - 1)
        def _():
            res_ref[...] = sums[...] / jnp.maximum(cnts[:, :1], 1.0)

    return pl.pallas_call(
        body,
        grid=(steps,),
        in_specs=[
            pl.BlockSpec((bn, _DF), lambda i: (i, 0)),
            pl.BlockSpec((1, 1, bn), lambda i: (i, 0, 0)),
        ],
        out_specs=pl.BlockSpec((_NGRAPH, _DH), lambda i: (0, 0)),
        out_shape=jax.ShapeDtypeStruct((_NGRAPH, _DH), jnp.float32),
        scratch_shapes=[
            pltpu.VMEM((_NGRAPH, _DH), jnp.float32),
            pltpu.VMEM((_NGRAPH, 128), jnp.float32),
        ],
    )(out2, batch3)


def _sc_layer(hl, hr, he, src, dst, perm, row_ptr, att):
    mesh = plsc.VectorSubcoreMesh(core_axis_name="c", subcore_axis_name="s")

    @functools.partial(
        pl.kernel, mesh=mesh,
        out_type=jax.ShapeDtypeStruct((_N, _DF), jnp.float32),
        scratch_types=[
            pltpu.VMEM((336,), jnp.int32),
            pltpu.SMEM((336,), jnp.int32),
            pltpu.VMEM((_CE,), jnp.int32),
            pltpu.VMEM((_CE,), jnp.int32),
            pltpu.VMEM((_CE,), jnp.int32),
            pltpu.VMEM((_CE, _DF), jnp.float32),
            pltpu.VMEM((_CE, _DF), jnp.float32),
            pltpu.VMEM((_CE, _DF), jnp.float32),
            pltpu.VMEM((_DF,), jnp.float32),
            pltpu.VMEM((_DF,), jnp.float32),
            pltpu.VMEM((1, _DF), jnp.float32),
            pltpu.SemaphoreType.DMA,
        ])
    def k(hl_hbm, hr_hbm, he_hbm, src_hbm, dst_hbm, perm_hbm, rp_hbm, att_hbm,
          out_hbm, rp_v, rp_s, perm_v, srcix, dstix, hlb, hrb, heb, attv,
          acc, orow, sem):
        wid = lax.axis_index("s") * 2 + lax.axis_index("c")
        n0 = wid * _NPT

        pltpu.sync_copy(att_hbm, attv)
        pltpu.sync_copy(rp_hbm.at[pl.ds(n0, 336)], rp_v)
        pltpu.sync_copy(rp_v, rp_s)

        te0 = rp_s[0]
        te1 = rp_s[_NPT]
        base0 = (te0 // 8) * 8
        nch = (te1 - base0 + _CE - 1) // _CE

        neg = jnp.full((16,), -1e30, jnp.float32)
        zero = jnp.zeros((16,), jnp.float32)

        def finalize(d, d0, d1):
            inv0 = 1.0 / (d0 + 1e-16)
            inv1 = 1.0 / (d1 + 1e-16)
            for j in range(64):
                sl = pl.ds(j * 16, 16)
                inv = inv0 if j < 32 else inv1
                den = d0 if j < 32 else d1
                orow[0, sl] = jnp.where(den > 0.0, acc[sl] * inv, 0.0)

            @pl.when(d < _N)
            def _():
                pltpu.sync_copy(orow, out_hbm.at[pl.ds(d, 1), :])

        def make_edge_body(base):
            def edge_body(local, car):
                e = base + local
                cd, m0, m1, d0, d1 = car

                def adv_cond(c):
                    return rp_s[c[0] + 1 - n0] <= e

                def adv_body(c):
                    cdi, _m0, _m1, _d0, _d1 = c
                    finalize(cdi, _d0, _d1)
                    return (cdi + 1, neg, neg, zero, zero)

                cd, m0, m1, d0, d1 = lax.while_loop(
                    adv_cond, adv_body, (cd, m0, m1, d0, d1))

                lg0 = zero
                lg1 = zero
                for j in range(64):
                    sl = pl.ds(j * 16, 16)
                    z = hlb[local, sl] + hrb[local, sl] + heb[local, sl]
                    z = jnp.maximum(z, 0.0) + 0.2 * jnp.minimum(z, 0.0)
                    if j < 32:
                        lg0 = lg0 + z * attv[sl]
                    else:
                        lg1 = lg1 + z * attv[sl]
                l0 = jnp.full((16,), jnp.sum(lg0), jnp.float32)
                l1 = jnp.full((16,), jnp.sum(lg1), jnp.float32)

                nm0 = jnp.maximum(m0, l0)
                nm1 = jnp.maximum(m1, l1)
                s0 = jnp.exp(m0 - nm0)
                s1 = jnp.exp(m1 - nm1)
                p0 = jnp.exp(l0 - nm0)
                p1 = jnp.exp(l1 - nm1)
                nd0 = d0 * s0 + p0
                nd1 = d1 * s1 + p1
                for j in range(64):
                    sl = pl.ds(j * 16, 16)
                    s = s0 if j < 32 else s1
                    p = p0 if j < 32 else p1
                    acc[sl] = acc[sl] * s + p * hlb[local, sl]
                return (cd, nm0, nm1, nd0, nd1)
            return edge_body

        def chunk_body(kk, car):
            base = base0 + kk * _CE
            pltpu.sync_copy(perm_hbm.at[pl.ds(base, _CE)], perm_v)
            pltpu.async_copy(src_hbm.at[perm_v], srcix, sem).wait()
            pltpu.async_copy(dst_hbm.at[perm_v], dstix, sem).wait()
            pltpu.async_copy(hl_hbm.at[srcix], hlb, sem).wait()
            pltpu.async_copy(hr_hbm.at[dstix], hrb, sem).wait()
            pltpu.async_copy(he_hbm.at[perm_v], heb, sem).wait()
            lo = jnp.maximum(te0 - base, 0)
            hi = jnp.minimum(te1 - base, _CE)
            return lax.fori_loop(lo, hi, make_edge_body(base), car)

        car = lax.fori_loop(0, nch, chunk_body, (n0, neg, neg, zero, zero))

        def tail_cond(c):
            return c[0] < n0 + _NPT

        def tail_body(c):
            cdi, _m0, _m1, _d0, _d1 = c
            finalize(cdi, _d0, _d1)
            return (cdi + 1, neg, neg, zero, zero)

        lax.while_loop(tail_cond, tail_body, car)

    return k(hl, hr, he, src, dst, perm, row_ptr, att)


def kernel(x, edge_index, edge_attr, batch, image, W_l1, W_r1, W_e1, att1,
           W_l2, W_r2, W_e2, att2):
    del image
    src = edge_index[0]
    dst = edge_index[1]

    dst_sorted, perm = lax.sort_key_val(dst, jnp.arange(_E, dtype=jnp.int32))
    row_ptr = jnp.searchsorted(
        dst_sorted, jnp.arange(_RP_PAD, dtype=jnp.int32), side="left"
    ).astype(jnp.int32)

    batch3 = batch.reshape(_N // 1000, 1, 1000)

    hl1, hr1 = _mm_node(x, W_l1, W_r1, fuse_head_mean=False)
    he1 = _mm_edge(edge_attr, W_e1)
    out1 = _sc_layer(hl1, hr1, he1, src, dst, perm, row_ptr,
                     att1.reshape(_DF))

    hl2, hr2 = _mm_node(out1, W_l2, W_r2, fuse_head_mean=True)
    he2 = _mm_edge(edge_attr, W_e2)
    out2 = _sc_layer(hl2, hr2, he2, src, dst, perm, row_ptr,
                     att2.reshape(_DF))

    return _pool(out2, batch3)


# speedup vs baseline: 4.9952x; 4.9952x over previous
import functools

import jax
import jax.numpy as jnp
from jax import lax
from jax.experimental import pallas as pl
from jax.experimental.pallas import tpu as pltpu
from jax.experimental.pallas import tpu_sc as plsc

_HEADS = 2
_DH = 512
_DF = _HEADS * _DH
_N = 10000
_E = 160000
_DIN = 256
_DEDGE = 7
_NGRAPH = 16

_NW = 32
_NPT = 320
_RP_PAD = 10256
_CE = 32


def _mm_node(h_in, W_l, W_r, fuse_head_mean):
    din = h_in.shape[1] // 2 if fuse_head_mean else h_in.shape[1]
    bn = 1000

    def body(h_ref, wl_ref, wr_ref, ol_ref, or_ref):
        hb = h_ref[...]
        if fuse_head_mean:
            hb = 0.5 * (hb[:, :din] + hb[:, din:])
        ol_ref[...] = jnp.dot(hb, wl_ref[...], preferred_element_type=jnp.float32)
        or_ref[...] = jnp.dot(hb, wr_ref[...], preferred_element_type=jnp.float32)

    return pl.pallas_call(
        body,
        grid=(_N // bn,),
        in_specs=[
            pl.BlockSpec((bn, h_in.shape[1]), lambda i: (i, 0)),
            pl.BlockSpec((din, _DF), lambda i: (0, 0)),
            pl.BlockSpec((din, _DF), lambda i: (0, 0)),
        ],
        out_specs=[
            pl.BlockSpec((bn, _DF), lambda i: (i, 0)),
            pl.BlockSpec((bn, _DF), lambda i: (i, 0)),
        ],
        out_shape=[
            jax.ShapeDtypeStruct((_N, _DF), jnp.float32),
            jax.ShapeDtypeStruct((_N, _DF), jnp.float32),
        ],
    )(h_in, W_l, W_r)


def _mm_edge(edge_attr, W_e):
    be = 4000

    def body(ea_ref, we_ref, o_ref):
        o_ref[...] = jnp.dot(ea_ref[...], we_ref[...],
                             preferred_element_type=jnp.float32)

    return pl.pallas_call(
        body,
        grid=(_E // be,),
        in_specs=[
            pl.BlockSpec((be, _DEDGE), lambda i: (i, 0)),
            pl.BlockSpec((_DEDGE, _DF), lambda i: (0, 0)),
        ],
        out_specs=pl.BlockSpec((be, _DF), lambda i: (i, 0)),
        out_shape=jax.ShapeDtypeStruct((_E, _DF), jnp.float32),
    )(edge_attr, W_e)


def _pool(out2, batch3):
    bn = 1000
    steps = _N // bn

    def body(o_ref, b_ref, res_ref, sums, cnts):
        i = pl.program_id(0)

        @pl.when(i == 0)
        def _():
            sums[...] = jnp.zeros_like(sums)
            cnts[...] = jnp.zeros_like(cnts)

        h = jnp.maximum(0.5 * (o_ref[:, :_DH] + o_ref[:, _DH:]), 0.0)
        b = b_ref[0, 0, :]
        gids = lax.broadcasted_iota(jnp.int32, (_NGRAPH, bn), 0)
        onehot = (gids == b[None, :]).astype(jnp.float32)
        sums[...] += jnp.dot(onehot, h, preferred_element_type=jnp.float32)
        cnts[...] += jnp.broadcast_to(
            jnp.sum(onehot, axis=1, keepdims=True), (_NGRAPH, 128))

        @pl.when(i == steps - 1)
        def _():
            res_ref[...] = sums[...] / jnp.maximum(cnts[:, :1], 1.0)

    return pl.pallas_call(
        body,
        grid=(steps,),
        in_specs=[
            pl.BlockSpec((bn, _DF), lambda i: (i, 0)),
            pl.BlockSpec((1, 1, bn), lambda i: (i, 0, 0)),
        ],
        out_specs=pl.BlockSpec((_NGRAPH, _DH), lambda i: (0, 0)),
        out_shape=jax.ShapeDtypeStruct((_NGRAPH, _DH), jnp.float32),
        scratch_shapes=[
            pltpu.VMEM((_NGRAPH, _DH), jnp.float32),
            pltpu.VMEM((_NGRAPH, 128), jnp.float32),
        ],
    )(out2, batch3)


def _sc_layer(hl, hr, he, src, dst, perm, row_ptr, att):
    mesh = plsc.VectorSubcoreMesh(core_axis_name="c", subcore_axis_name="s")

    def lane_sum(v):
        ii = lax.broadcasted_iota(jnp.int32, (16,), 0)
        dnums = lax.GatherDimensionNumbers(
            offset_dims=(), collapsed_slice_dims=(0,), start_index_map=(0,))
        for k in (8, 4, 2, 1):
            v = v + lax.gather(
                v, (ii ^ k)[:, None], dnums, slice_sizes=(1,),
                mode=lax.GatherScatterMode.PROMISE_IN_BOUNDS)
        return v

    @functools.partial(
        pl.kernel, mesh=mesh,
        out_type=jax.ShapeDtypeStruct((_N, _DF), jnp.float32),
        scratch_types=[
            pltpu.VMEM((336,), jnp.int32),
            pltpu.VMEM((_CE,), jnp.int32),
            pltpu.VMEM((_CE,), jnp.int32),
            pltpu.VMEM((_CE,), jnp.int32),
            pltpu.VMEM((_CE + 16,), jnp.int32),
            pltpu.VMEM((_CE, _DF), jnp.float32),
            pltpu.VMEM((_CE, _DF), jnp.float32),
            pltpu.VMEM((_CE, _DF), jnp.float32),
            pltpu.VMEM((_DF,), jnp.float32),
            pltpu.VMEM((_DF,), jnp.float32),
            pltpu.VMEM((1, _DF), jnp.float32),
            pltpu.VMEM((8, _DF), jnp.float32),
            pltpu.SemaphoreType.DMA,
        ])
    def k(hl_hbm, hr_hbm, he_hbm, srcs_hbm, dsts_hbm, perm_hbm, rp_hbm,
          att_hbm, out_hbm, rp_v, perm_v, srcix, dstg, dstix, hlb, hrb, heb,
          attv, acc, orow, zblk, sem):
        wid = lax.axis_index("s") * 2 + lax.axis_index("c")
        n0 = wid * _NPT
        n_hi = jnp.minimum(n0 + _NPT, _N)

        pltpu.sync_copy(att_hbm, attv)
        pltpu.sync_copy(rp_hbm.at[pl.ds(n0, 336)], rp_v)

        te0 = rp_v[pl.ds(0, 16)][0]
        te1 = rp_v[pl.ds(_NPT, 16)][0]
        base0 = (te0 // 8) * 8
        nch = (te1 - base0 + _CE - 1) // _CE

        neg = jnp.full((16,), -1e30, jnp.float32)
        zero = jnp.zeros((16,), jnp.float32)

        def zrow_body(j, c):
            zblk[j // 64, pl.ds((j % 64) * 16, 16)] = zero
            return c

        lax.fori_loop(0, 8 * 64, zrow_body, 0, unroll=8)

        def zfill_body(i, carry):
            pltpu.sync_copy(zblk, out_hbm.at[pl.ds(n0 + i * 8, 8), :])
            return carry

        lax.fori_loop(0, (n_hi - n0) // 8, zfill_body, 0)

        def finalize(d, d0, d1):
            inv0 = 1.0 / (d0 + 1e-16)
            inv1 = 1.0 / (d1 + 1e-16)

            def fin_body(inv):
                def b(j, c):
                    sl = pl.ds(j * 16, 16)
                    orow[0, sl] = acc[sl] * inv
                    return c
                return b

            lax.fori_loop(0, 32, fin_body(inv0), 0, unroll=8)
            lax.fori_loop(32, 64, fin_body(inv1), 0, unroll=8)
            pltpu.sync_copy(orow, out_hbm.at[pl.ds(d, 1), :])

        def make_edge_body(base):
            def edge_body(local, car):
                cd, m0, m1, d0, d1 = car
                d_e = dstix[pl.ds(local, 16)][0]
                is_new = d_e != cd

                @pl.when(is_new & (cd >= 0))
                def _():
                    finalize(cd, d0, d1)

                m0 = jnp.where(is_new, neg, m0)
                m1 = jnp.where(is_new, neg, m1)
                d0 = jnp.where(is_new, zero, d0)
                d1 = jnp.where(is_new, zero, d1)

                def lg_body(j, a):
                    sl = pl.ds(j * 16, 16)
                    z = hlb[local, sl] + hrb[local, sl] + heb[local, sl]
                    z = jnp.maximum(z, 0.0) + 0.2 * jnp.minimum(z, 0.0)
                    return a + z * attv[sl]

                lg0 = lax.fori_loop(0, 32, lg_body, zero, unroll=8)
                lg1 = lax.fori_loop(32, 64, lg_body, zero, unroll=8)
                l0 = lane_sum(lg0)
                l1 = lane_sum(lg1)

                nm0 = jnp.maximum(m0, l0)
                nm1 = jnp.maximum(m1, l1)
                s0 = jnp.exp(m0 - nm0)
                s1 = jnp.exp(m1 - nm1)
                p0 = jnp.exp(l0 - nm0)
                p1 = jnp.exp(l1 - nm1)
                nd0 = d0 * s0 + p0
                nd1 = d1 * s1 + p1

                def acc_body(s, p):
                    def b(j, c):
                        sl = pl.ds(j * 16, 16)
                        acc[sl] = acc[sl] * s + p * hlb[local, sl]
                        return c
                    return b

                lax.fori_loop(0, 32, acc_body(s0, p0), 0, unroll=8)
                lax.fori_loop(32, 64, acc_body(s1, p1), 0, unroll=8)
                return (d_e, nm0, nm1, nd0, nd1)
            return edge_body

        def chunk_body(kk, car):
            base = base0 + kk * _CE
            pltpu.sync_copy(perm_hbm.at[pl.ds(base, _CE)], perm_v)
            pltpu.sync_copy(srcs_hbm.at[pl.ds(base, _CE)], srcix)
            pltpu.sync_copy(dsts_hbm.at[pl.ds(base, _CE)], dstg)
            pltpu.async_copy(hl_hbm.at[srcix], hlb, sem).wait()
            pltpu.async_copy(hr_hbm.at[dstg], hrb, sem).wait()
            for j in range(_CE // 16):
                dstix[pl.ds(j * 16, 16)] = dstg[pl.ds(j * 16, 16)]
            pltpu.async_copy(he_hbm.at[perm_v], heb, sem).wait()
            lo = jnp.maximum(te0 - base, 0)
            hi = jnp.maximum(jnp.minimum(te1 - base, _CE), lo)
            return lax.fori_loop(lo, hi, make_edge_body(base), car)

        cd, m0, m1, d0, d1 = lax.fori_loop(
            0, nch, chunk_body, (jnp.int32(-1), neg, neg, zero, zero))

        @pl.when(cd >= 0)
        def _():
            finalize(cd, d0, d1)

    return k(hl, hr, he, src, dst, perm, row_ptr, att)


def kernel(x, edge_index, edge_attr, batch, image, W_l1, W_r1, W_e1, att1,
           W_l2, W_r2, W_e2, att2):
    del image
    src = edge_index[0]
    dst = edge_index[1]

    dst_sorted, perm = lax.sort_key_val(dst, jnp.arange(_E, dtype=jnp.int32))
    row_ptr = jnp.searchsorted(
        dst_sorted, jnp.arange(_RP_PAD, dtype=jnp.int32), side="left"
    ).astype(jnp.int32)
    pad = jnp.zeros((_CE * 2,), jnp.int32)
    src_s = jnp.concatenate([src[perm], pad])
    dst_s = jnp.concatenate([dst_sorted, pad])
    perm_p = jnp.concatenate([perm, pad])

    batch3 = batch.reshape(_N // 1000, 1, 1000)

    hl1, hr1 = _mm_node(x, W_l1, W_r1, fuse_head_mean=False)
    he1 = _mm_edge(edge_attr, W_e1)
    out1 = _sc_layer(hl1, hr1, he1, src_s, dst_s, perm_p, row_ptr,
                     att1.reshape(_DF))

    hl2, hr2 = _mm_node(out1, W_l2, W_r2, fuse_head_mean=True)
    he2 = _mm_edge(edge_attr, W_e2)
    out2 = _sc_layer(hl2, hr2, he2, src_s, dst_s, perm_p, row_ptr,
                     att2.reshape(_DF))

    return _pool(out2, batch3)


# speedup vs baseline: 5.8170x; 1.1645x over previous
import functools

import jax
import jax.numpy as jnp
from jax import lax
from jax.experimental import pallas as pl
from jax.experimental.pallas import tpu as pltpu
from jax.experimental.pallas import tpu_sc as plsc

_HEADS = 2
_DH = 512
_DF = _HEADS * _DH
_N = 10000
_E = 160000
_DIN = 256
_DEDGE = 7
_NGRAPH = 16

_NW = 32
_NPT = 320
_RP_PAD = 10256
_CE = 32


def _mm_node(h_in, W_l, W_r, fuse_head_mean):
    din = h_in.shape[1] // 2 if fuse_head_mean else h_in.shape[1]
    bn = 1000

    def body(h_ref, wl_ref, wr_ref, ol_ref, or_ref):
        hb = h_ref[...]
        if fuse_head_mean:
            hb = 0.5 * (hb[:, :din] + hb[:, din:])
        ol_ref[...] = jnp.dot(hb, wl_ref[...], preferred_element_type=jnp.float32)
        or_ref[...] = jnp.dot(hb, wr_ref[...], preferred_element_type=jnp.float32)

    return pl.pallas_call(
        body,
        grid=(_N // bn,),
        in_specs=[
            pl.BlockSpec((bn, h_in.shape[1]), lambda i: (i, 0)),
            pl.BlockSpec((din, _DF), lambda i: (0, 0)),
            pl.BlockSpec((din, _DF), lambda i: (0, 0)),
        ],
        out_specs=[
            pl.BlockSpec((bn, _DF), lambda i: (i, 0)),
            pl.BlockSpec((bn, _DF), lambda i: (i, 0)),
        ],
        out_shape=[
            jax.ShapeDtypeStruct((_N, _DF), jnp.float32),
            jax.ShapeDtypeStruct((_N, _DF), jnp.float32),
        ],
    )(h_in, W_l, W_r)


def _mm_edge(edge_attr, W_e):
    be = 4000

    def body(ea_ref, we_ref, o_ref):
        o_ref[...] = jnp.dot(ea_ref[...], we_ref[...],
                             preferred_element_type=jnp.float32)

    return pl.pallas_call(
        body,
        grid=(_E // be,),
        in_specs=[
            pl.BlockSpec((be, _DEDGE), lambda i: (i, 0)),
            pl.BlockSpec((_DEDGE, _DF), lambda i: (0, 0)),
        ],
        out_specs=pl.BlockSpec((be, _DF), lambda i: (i, 0)),
        out_shape=jax.ShapeDtypeStruct((_E, _DF), jnp.float32),
    )(edge_attr, W_e)


def _pool(out2, batch3):
    bn = 1000
    steps = _N // bn

    def body(o_ref, b_ref, res_ref, sums, cnts):
        i = pl.program_id(0)

        @pl.when(i == 0)
        def _():
            sums[...] = jnp.zeros_like(sums)
            cnts[...] = jnp.zeros_like(cnts)

        h = jnp.maximum(0.5 * (o_ref[:, :_DH] + o_ref[:, _DH:]), 0.0)
        b = b_ref[0, 0, :]
        gids = lax.broadcasted_iota(jnp.int32, (_NGRAPH, bn), 0)
        onehot = (gids == b[None, :]).astype(jnp.float32)
        sums[...] += jnp.dot(onehot, h, preferred_element_type=jnp.float32)
        cnts[...] += jnp.broadcast_to(
            jnp.sum(onehot, axis=1, keepdims=True), (_NGRAPH, 128))

        @pl.when(i == steps - 1)
        def _():
            res_ref[...] = sums[...] / jnp.maximum(cnts[:, :1], 1.0)

    return pl.pallas_call(
        body,
        grid=(steps,),
        in_specs=[
            pl.BlockSpec((bn, _DF), lambda i: (i, 0)),
            pl.BlockSpec((1, 1, bn), lambda i: (i, 0, 0)),
        ],
        out_specs=pl.BlockSpec((_NGRAPH, _DH), lambda i: (0, 0)),
        out_shape=jax.ShapeDtypeStruct((_NGRAPH, _DH), jnp.float32),
        scratch_shapes=[
            pltpu.VMEM((_NGRAPH, _DH), jnp.float32),
            pltpu.VMEM((_NGRAPH, 128), jnp.float32),
        ],
    )(out2, batch3)


def _sc_layer(hl, hr, he, src, dst, perm, row_ptr, att):
    mesh = plsc.VectorSubcoreMesh(core_axis_name="c", subcore_axis_name="s")

    def lane_sum(v):
        ii = lax.broadcasted_iota(jnp.int32, (16,), 0)
        dnums = lax.GatherDimensionNumbers(
            offset_dims=(), collapsed_slice_dims=(0,), start_index_map=(0,))
        for k in (8, 4, 2, 1):
            v = v + lax.gather(
                v, (ii ^ k)[:, None], dnums, slice_sizes=(1,),
                mode=lax.GatherScatterMode.PROMISE_IN_BOUNDS)
        return v

    @functools.partial(
        pl.kernel, mesh=mesh,
        out_type=jax.ShapeDtypeStruct((_N, _DF), jnp.float32),
        scratch_types=[
            pltpu.VMEM((336,), jnp.int32),
            pltpu.VMEM((_CE,), jnp.int32),
            pltpu.VMEM((_CE,), jnp.int32),
            pltpu.VMEM((_CE,), jnp.int32),
            pltpu.VMEM((_CE + 16,), jnp.int32),
            pltpu.VMEM((_CE, _DF), jnp.float32),
            pltpu.VMEM((16, _DF), jnp.float32),
            pltpu.VMEM((_CE, _DF), jnp.float32),
            pltpu.VMEM((_DF,), jnp.float32),
            pltpu.VMEM((_DF,), jnp.float32),
            pltpu.VMEM((1, _DF), jnp.float32),
            pltpu.VMEM((8, _DF), jnp.float32),
            pltpu.SemaphoreType.DMA,
        ])
    def k(hl_hbm, hr_hbm, he_hbm, srcs_hbm, dsts_hbm, perm_hbm, rp_hbm,
          att_hbm, out_hbm, rp_v, perm_v, srcix, dstg, dstix, hlb, hrw, heb,
          attv, acc, orow, zblk, sem):
        wid = lax.axis_index("s") * 2 + lax.axis_index("c")
        n0 = wid * _NPT
        n_hi = jnp.minimum(n0 + _NPT, _N)

        pltpu.sync_copy(att_hbm, attv)
        pltpu.sync_copy(rp_hbm.at[pl.ds(n0, 336)], rp_v)

        te0 = rp_v[pl.ds(0, 16)][0]
        te1 = rp_v[pl.ds(_NPT, 16)][0]
        base0 = (te0 // 8) * 8
        nch = (te1 - base0 + _CE - 1) // _CE

        neg = jnp.full((16,), -1e30, jnp.float32)
        zero = jnp.zeros((16,), jnp.float32)

        def zrow_body(j, c):
            zblk[j // 64, pl.ds((j % 64) * 16, 16)] = zero
            return c

        lax.fori_loop(0, 8 * 64, zrow_body, 0, unroll=8)

        def zfill_body(i, carry):
            pltpu.sync_copy(zblk, out_hbm.at[pl.ds(n0 + i * 8, 8), :])
            return carry

        lax.fori_loop(0, (n_hi - n0) // 8, zfill_body, 0)

        def finalize(d, d0, d1):
            inv0 = 1.0 / (d0 + 1e-16)
            inv1 = 1.0 / (d1 + 1e-16)

            def fin_body(inv):
                def b(j, c):
                    sl = pl.ds(j * 16, 16)
                    orow[0, sl] = acc[sl] * inv
                    return c
                return b

            lax.fori_loop(0, 32, fin_body(inv0), 0, unroll=8)
            lax.fori_loop(32, 64, fin_body(inv1), 0, unroll=8)
            pltpu.sync_copy(orow, out_hbm.at[pl.ds(d, 1), :])

        def make_edge_body(base):
            def edge_body(local, car):
                cd, wb, m0, m1, d0, d1 = car
                d_e = dstix[pl.ds(local, 16)][0]
                is_new = d_e != cd

                @pl.when(is_new & (cd >= 0))
                def _():
                    finalize(cd, d0, d1)

                adv = d_e >= wb + 16
                wb = jnp.where(adv, n0 + ((d_e - n0) // 16) * 16, wb)

                @pl.when(adv)
                def _():
                    pltpu.sync_copy(
                        hr_hbm.at[pl.ds(pl.multiple_of(wb, 16), 16), :], hrw)

                roff = d_e - wb
                m0 = jnp.where(is_new, neg, m0)
                m1 = jnp.where(is_new, neg, m1)
                d0 = jnp.where(is_new, zero, d0)
                d1 = jnp.where(is_new, zero, d1)

                def lg_body(j, a):
                    sl = pl.ds(j * 16, 16)
                    z = hlb[local, sl] + hrw[roff, sl] + heb[local, sl]
                    z = jnp.maximum(z, 0.0) + 0.2 * jnp.minimum(z, 0.0)
                    return a + z * attv[sl]

                lg0 = lax.fori_loop(0, 32, lg_body, zero, unroll=8)
                lg1 = lax.fori_loop(32, 64, lg_body, zero, unroll=8)
                l0 = lane_sum(lg0)
                l1 = lane_sum(lg1)

                nm0 = jnp.maximum(m0, l0)
                nm1 = jnp.maximum(m1, l1)
                s0 = jnp.exp(m0 - nm0)
                s1 = jnp.exp(m1 - nm1)
                p0 = jnp.exp(l0 - nm0)
                p1 = jnp.exp(l1 - nm1)
                nd0 = d0 * s0 + p0
                nd1 = d1 * s1 + p1

                def acc_body(s, p):
                    def b(j, c):
                        sl = pl.ds(j * 16, 16)
                        acc[sl] = acc[sl] * s + p * hlb[local, sl]
                        return c
                    return b

                lax.fori_loop(0, 32, acc_body(s0, p0), 0, unroll=8)
                lax.fori_loop(32, 64, acc_body(s1, p1), 0, unroll=8)
                return (d_e, wb, nm0, nm1, nd0, nd1)
            return edge_body

        def chunk_body(kk, car):
            base = base0 + kk * _CE
            c1 = pltpu.async_copy(perm_hbm.at[pl.ds(base, _CE)], perm_v, sem)
            c2 = pltpu.async_copy(srcs_hbm.at[pl.ds(base, _CE)], srcix, sem)
            c3 = pltpu.async_copy(dsts_hbm.at[pl.ds(base, _CE)], dstg, sem)
            c1.wait()
            c2.wait()
            c3.wait()
            g1 = pltpu.async_copy(hl_hbm.at[srcix], hlb, sem)
            g2 = pltpu.async_copy(he_hbm.at[perm_v], heb, sem)
            for j in range(_CE // 16):
                dstix[pl.ds(j * 16, 16)] = dstg[pl.ds(j * 16, 16)]
            g1.wait()
            g2.wait()
            lo = jnp.maximum(te0 - base, 0)
            hi = jnp.maximum(jnp.minimum(te1 - base, _CE), lo)
            return lax.fori_loop(lo, hi, make_edge_body(base), car)

        cd, _wb, m0, m1, d0, d1 = lax.fori_loop(
            0, nch, chunk_body,
            (jnp.int32(-1), jnp.int32(-2**30), neg, neg, zero, zero))

        @pl.when(cd >= 0)
        def _():
            finalize(cd, d0, d1)

    return k(hl, hr, he, src, dst, perm, row_ptr, att)


def kernel(x, edge_index, edge_attr, batch, image, W_l1, W_r1, W_e1, att1,
           W_l2, W_r2, W_e2, att2):
    del image
    src = edge_index[0]
    dst = edge_index[1]

    dst_sorted, perm = lax.sort_key_val(dst, jnp.arange(_E, dtype=jnp.int32))
    row_ptr = jnp.searchsorted(
        dst_sorted, jnp.arange(_RP_PAD, dtype=jnp.int32), side="left"
    ).astype(jnp.int32)
    pad = jnp.zeros((_CE * 2,), jnp.int32)
    src_s = jnp.concatenate([src[perm], pad])
    dst_s = jnp.concatenate([dst_sorted, pad])
    perm_p = jnp.concatenate([perm, pad])

    batch3 = batch.reshape(_N // 1000, 1, 1000)

    hl1, hr1 = _mm_node(x, W_l1, W_r1, fuse_head_mean=False)
    he1 = _mm_edge(edge_attr, W_e1)
    out1 = _sc_layer(hl1, hr1, he1, src_s, dst_s, perm_p, row_ptr,
                     att1.reshape(_DF))

    hl2, hr2 = _mm_node(out1, W_l2, W_r2, fuse_head_mean=True)
    he2 = _mm_edge(edge_attr, W_e2)
    out2 = _sc_layer(hl2, hr2, he2, src_s, dst_s, perm_p, row_ptr,
                     att2.reshape(_DF))

    return _pool(out2, batch3)


# speedup vs baseline: 6.6708x; 1.1468x over previous
import functools

import jax
import jax.numpy as jnp
from jax import lax
from jax.experimental import pallas as pl
from jax.experimental.pallas import tpu as pltpu
from jax.experimental.pallas import tpu_sc as plsc

_HEADS = 2
_DH = 512
_DF = _HEADS * _DH
_N = 10000
_E = 160000
_DIN = 256
_DEDGE = 7
_NGRAPH = 16

_NW = 32
_NPT = 320
_RP_PAD = 10256
_CE = 16


def _mm_node(h_in, W_l, W_r, fuse_head_mean):
    din = h_in.shape[1] // 2 if fuse_head_mean else h_in.shape[1]
    bn = 1000

    def body(h_ref, wl_ref, wr_ref, ol_ref, or_ref):
        hb = h_ref[...]
        if fuse_head_mean:
            hb = 0.5 * (hb[:, :din] + hb[:, din:])
        ol_ref[...] = jnp.dot(hb, wl_ref[...], preferred_element_type=jnp.float32)
        or_ref[...] = jnp.dot(hb, wr_ref[...], preferred_element_type=jnp.float32)

    return pl.pallas_call(
        body,
        grid=(_N // bn,),
        in_specs=[
            pl.BlockSpec((bn, h_in.shape[1]), lambda i: (i, 0)),
            pl.BlockSpec((din, _DF), lambda i: (0, 0)),
            pl.BlockSpec((din, _DF), lambda i: (0, 0)),
        ],
        out_specs=[
            pl.BlockSpec((bn, _DF), lambda i: (i, 0)),
            pl.BlockSpec((bn, _DF), lambda i: (i, 0)),
        ],
        out_shape=[
            jax.ShapeDtypeStruct((_N, _DF), jnp.float32),
            jax.ShapeDtypeStruct((_N, _DF), jnp.float32),
        ],
    )(h_in, W_l, W_r)


def _mm_edge(edge_attr, W_e):
    be = 4000

    def body(ea_ref, we_ref, o_ref):
        o_ref[...] = jnp.dot(ea_ref[...], we_ref[...],
                             preferred_element_type=jnp.float32)

    return pl.pallas_call(
        body,
        grid=(_E // be,),
        in_specs=[
            pl.BlockSpec((be, _DEDGE), lambda i: (i, 0)),
            pl.BlockSpec((_DEDGE, _DF), lambda i: (0, 0)),
        ],
        out_specs=pl.BlockSpec((be, _DF), lambda i: (i, 0)),
        out_shape=jax.ShapeDtypeStruct((_E, _DF), jnp.float32),
    )(edge_attr, W_e)


def _pool(out2, batch3):
    bn = 1000
    steps = _N // bn

    def body(o_ref, b_ref, res_ref, sums, cnts):
        i = pl.program_id(0)

        @pl.when(i == 0)
        def _():
            sums[...] = jnp.zeros_like(sums)
            cnts[...] = jnp.zeros_like(cnts)

        h = jnp.maximum(0.5 * (o_ref[:, :_DH] + o_ref[:, _DH:]), 0.0)
        b = b_ref[0, 0, :]
        gids = lax.broadcasted_iota(jnp.int32, (_NGRAPH, bn), 0)
        onehot = (gids == b[None, :]).astype(jnp.float32)
        sums[...] += jnp.dot(onehot, h, preferred_element_type=jnp.float32)
        cnts[...] += jnp.broadcast_to(
            jnp.sum(onehot, axis=1, keepdims=True), (_NGRAPH, 128))

        @pl.when(i == steps - 1)
        def _():
            res_ref[...] = sums[...] / jnp.maximum(cnts[:, :1], 1.0)

    return pl.pallas_call(
        body,
        grid=(steps,),
        in_specs=[
            pl.BlockSpec((bn, _DF), lambda i: (i, 0)),
            pl.BlockSpec((1, 1, bn), lambda i: (i, 0, 0)),
        ],
        out_specs=pl.BlockSpec((_NGRAPH, _DH), lambda i: (0, 0)),
        out_shape=jax.ShapeDtypeStruct((_NGRAPH, _DH), jnp.float32),
        scratch_shapes=[
            pltpu.VMEM((_NGRAPH, _DH), jnp.float32),
            pltpu.VMEM((_NGRAPH, 128), jnp.float32),
        ],
    )(out2, batch3)


def _sc_layer(hl, hr, he, src, dst, perm, row_ptr, att):
    mesh = plsc.VectorSubcoreMesh(core_axis_name="c", subcore_axis_name="s")

    def lane_sum(v):
        ii = lax.broadcasted_iota(jnp.int32, (16,), 0)
        dnums = lax.GatherDimensionNumbers(
            offset_dims=(), collapsed_slice_dims=(0,), start_index_map=(0,))
        for k in (8, 4, 2, 1):
            v = v + lax.gather(
                v, (ii ^ k)[:, None], dnums, slice_sizes=(1,),
                mode=lax.GatherScatterMode.PROMISE_IN_BOUNDS)
        return v

    @functools.partial(
        pl.kernel, mesh=mesh,
        out_type=jax.ShapeDtypeStruct((_N, _DF), jnp.float32),
        scratch_types=[
            pltpu.VMEM((336,), jnp.int32),
            pltpu.VMEM((_CE,), jnp.int32),
            pltpu.VMEM((_CE,), jnp.int32),
            pltpu.VMEM((_CE,), jnp.int32),
            pltpu.VMEM((_CE + 16,), jnp.int32),
            pltpu.VMEM((_CE, _DF), jnp.float32),
            pltpu.VMEM((_CE, _DF), jnp.float32),
            pltpu.VMEM((_CE,), jnp.int32),
            pltpu.VMEM((_CE,), jnp.int32),
            pltpu.VMEM((_CE,), jnp.int32),
            pltpu.VMEM((_CE + 16,), jnp.int32),
            pltpu.VMEM((_CE, _DF), jnp.float32),
            pltpu.VMEM((_CE, _DF), jnp.float32),
            pltpu.VMEM((16, _DF), jnp.float32),
            pltpu.VMEM((_DF,), jnp.float32),
            pltpu.VMEM((_DF,), jnp.float32),
            pltpu.VMEM((1, _DF), jnp.float32),
            pltpu.VMEM((8, _DF), jnp.float32),
            pltpu.SemaphoreType.DMA,
            pltpu.SemaphoreType.DMA,
            pltpu.SemaphoreType.DMA,
            pltpu.SemaphoreType.DMA,
        ])
    def k(hl_hbm, hr_hbm, he_hbm, srcs_hbm, dsts_hbm, perm_hbm, rp_hbm,
          att_hbm, out_hbm, rp_v,
          perm0, src0, dst0, dstix0, hlb0, heb0,
          perm1, src1, dst1, dstix1, hlb1, heb1,
          hrw, attv, acc, orow, zblk, isem0, isem1, gsem0, gsem1):
        permb = (perm0, perm1)
        srcb = (src0, src1)
        dstb = (dst0, dst1)
        dstixb = (dstix0, dstix1)
        hlbb = (hlb0, hlb1)
        hebb = (heb0, heb1)
        isem = (isem0, isem1)
        gsem = (gsem0, gsem1)
        wid = lax.axis_index("s") * 2 + lax.axis_index("c")
        n0 = wid * _NPT
        n_hi = jnp.minimum(n0 + _NPT, _N)

        pltpu.sync_copy(att_hbm, attv)
        pltpu.sync_copy(rp_hbm.at[pl.ds(n0, 336)], rp_v)

        te0 = rp_v[pl.ds(0, 16)][0]
        te1 = rp_v[pl.ds(_NPT, 16)][0]
        base0 = (te0 // 8) * 8
        nch = (te1 - base0 + _CE - 1) // _CE

        neg = jnp.full((16,), -1e30, jnp.float32)
        zero = jnp.zeros((16,), jnp.float32)

        def zrow_body(j, c):
            zblk[j // 64, pl.ds((j % 64) * 16, 16)] = zero
            return c

        lax.fori_loop(0, 8 * 64, zrow_body, 0, unroll=8)

        def zfill_body(i, carry):
            pltpu.sync_copy(zblk, out_hbm.at[pl.ds(n0 + i * 8, 8), :])
            return carry

        lax.fori_loop(0, (n_hi - n0) // 8, zfill_body, 0)

        def finalize(d, d0, d1):
            inv0 = 1.0 / (d0 + 1e-16)
            inv1 = 1.0 / (d1 + 1e-16)

            def fin_body(inv):
                def b(j, c):
                    sl = pl.ds(j * 16, 16)
                    orow[0, sl] = acc[sl] * inv
                    return c
                return b

            lax.fori_loop(0, 32, fin_body(inv0), 0, unroll=8)
            lax.fori_loop(32, 64, fin_body(inv1), 0, unroll=8)
            pltpu.sync_copy(orow, out_hbm.at[pl.ds(d, 1), :])

        def make_edge_body(base, dstix, hlb, heb):
            def edge_body(local, car):
                cd, wb, m0, m1, d0, d1 = car
                d_e = dstix[pl.ds(local, 16)][0]
                is_new = d_e != cd

                @pl.when(is_new & (cd >= 0))
                def _():
                    finalize(cd, d0, d1)

                adv = d_e >= wb + 16
                wb = jnp.where(adv, n0 + ((d_e - n0) // 16) * 16, wb)

                @pl.when(adv)
                def _():
                    pltpu.sync_copy(
                        hr_hbm.at[pl.ds(pl.multiple_of(wb, 16), 16), :], hrw)

                roff = d_e - wb
                m0 = jnp.where(is_new, neg, m0)
                m1 = jnp.where(is_new, neg, m1)
                d0 = jnp.where(is_new, zero, d0)
                d1 = jnp.where(is_new, zero, d1)

                def lg_body(j, a):
                    sl = pl.ds(j * 16, 16)
                    z = hlb[local, sl] + hrw[roff, sl] + heb[local, sl]
                    z = jnp.maximum(z, 0.0) + 0.2 * jnp.minimum(z, 0.0)
                    return a + z * attv[sl]

                lg0 = lax.fori_loop(0, 32, lg_body, zero, unroll=8)
                lg1 = lax.fori_loop(32, 64, lg_body, zero, unroll=8)
                l0 = lane_sum(lg0)
                l1 = lane_sum(lg1)

                nm0 = jnp.maximum(m0, l0)
                nm1 = jnp.maximum(m1, l1)
                s0 = jnp.exp(m0 - nm0)
                s1 = jnp.exp(m1 - nm1)
                p0 = jnp.exp(l0 - nm0)
                p1 = jnp.exp(l1 - nm1)
                nd0 = d0 * s0 + p0
                nd1 = d1 * s1 + p1

                def acc_body(s, p):
                    def b(j, c):
                        sl = pl.ds(j * 16, 16)
                        acc[sl] = acc[sl] * s + p * hlb[local, sl]
                        return c
                    return b

                lax.fori_loop(0, 32, acc_body(s0, p0), 0, unroll=8)
                lax.fori_loop(32, 64, acc_body(s1, p1), 0, unroll=8)
                return (d_e, wb, nm0, nm1, nd0, nd1)
            return edge_body

        def fire_idx(k, s):
            base = base0 + k * _CE
            pltpu.async_copy(perm_hbm.at[pl.ds(base, _CE)], permb[s], isem[s])
            pltpu.async_copy(srcs_hbm.at[pl.ds(base, _CE)], srcb[s], isem[s])
            pltpu.async_copy(dsts_hbm.at[pl.ds(base, _CE)], dstb[s], isem[s])

        def drain_idx(s):
            for buf in (permb[s], srcb[s], dstb[s]):
                pltpu.make_async_copy(
                    perm_hbm.at[pl.ds(0, _CE)], buf, isem[s]).wait()
            dstixb[s][pl.ds(0, 16)] = dstb[s][pl.ds(0, 16)]

        def fire_rows(s):
            pltpu.async_copy(hl_hbm.at[srcb[s]], hlbb[s], gsem[s])
            pltpu.async_copy(he_hbm.at[permb[s]], hebb[s], gsem[s])

        def drain_rows(s):
            pltpu.make_async_copy(
                hl_hbm.at[pl.ds(0, _CE), :], hlbb[s], gsem[s]).wait()
            pltpu.make_async_copy(
                hl_hbm.at[pl.ds(0, _CE), :], hebb[s], gsem[s]).wait()

        def compute(k, s, car):
            base = base0 + k * _CE
            lo = jnp.maximum(te0 - base, 0)
            hi = jnp.maximum(jnp.minimum(te1 - base, _CE), lo)
            return lax.fori_loop(
                lo, hi, make_edge_body(base, dstixb[s], hlbb[s], hebb[s]),
                car)

        fire_idx(0, 0)
        drain_idx(0)
        fire_rows(0)
        fire_idx(1, 1)

        def pair_body(kk, car):
            k0 = 2 * kk
            drain_rows(0)
            drain_idx(1)
            fire_rows(1)
            fire_idx(k0 + 2, 0)
            car = compute(k0, 0, car)
            drain_rows(1)
            drain_idx(0)
            fire_rows(0)
            fire_idx(k0 + 3, 1)
            car = compute(k0 + 1, 1, car)
            return car

        npairs = (nch + 1) // 2
        cd, _wb, m0, m1, d0, d1 = lax.fori_loop(
            0, npairs, pair_body,
            (jnp.int32(-1), jnp.int32(-2**30), neg, neg, zero, zero))

        drain_rows(0)
        drain_idx(1)

        @pl.when(cd >= 0)
        def _():
            finalize(cd, d0, d1)

    return k(hl, hr, he, src, dst, perm, row_ptr, att)


def kernel(x, edge_index, edge_attr, batch, image, W_l1, W_r1, W_e1, att1,
           W_l2, W_r2, W_e2, att2):
    del image
    src = edge_index[0]
    dst = edge_index[1]

    dst_sorted, perm = lax.sort_key_val(dst, jnp.arange(_E, dtype=jnp.int32))
    row_ptr = jnp.searchsorted(
        dst_sorted, jnp.arange(_RP_PAD, dtype=jnp.int32), side="left"
    ).astype(jnp.int32)
    pad = jnp.zeros((_CE * 8,), jnp.int32)
    src_s = jnp.concatenate([src[perm], pad])
    dst_s = jnp.concatenate([dst_sorted, pad])
    perm_p = jnp.concatenate([perm, pad])

    batch3 = batch.reshape(_N // 1000, 1, 1000)

    hl1, hr1 = _mm_node(x, W_l1, W_r1, fuse_head_mean=False)
    he1 = _mm_edge(edge_attr, W_e1)
    out1 = _sc_layer(hl1, hr1, he1, src_s, dst_s, perm_p, row_ptr,
                     att1.reshape(_DF))

    hl2, hr2 = _mm_node(out1, W_l2, W_r2, fuse_head_mean=True)
    he2 = _mm_edge(edge_attr, W_e2)
    out2 = _sc_layer(hl2, hr2, he2, src_s, dst_s, perm_p, row_ptr,
                     att2.reshape(_DF))

    return _pool(out2, batch3)
